# baseline (speedup 1.0000x reference)
import functools

import jax
import jax.numpy as jnp
from jax import lax
from jax.experimental import pallas as pl
from jax.experimental.pallas import tpu as pltpu
from jax.experimental.pallas import tpu_sc as plsc

NC = 2
NS = 16
NW = NC * NS

B, T, F = 16, 1024, 768
TPW = T // NC
CHUNK = 16
NCH = TPW // CHUNK
NB = 8
SUB = 8
VL = 16

_K2 = -2.0 * 0.7978845608028654
_A = 0.044715


def _gelu_vec(x):
    x2 = x * x
    arg = (_K2 * x) * (1.0 + _A * x2)
    return x / (1.0 + jnp.exp(arg))


def _sc_body(hid, idx, out, bufs, mask_v, idx_v, gsems, ssems):
    wid = lax.axis_index("s") * NC + lax.axis_index("c")
    batch = wid // NC
    t0 = (wid % NC) * TPW

    def g_copy(c, q):
        return pltpu.make_async_copy(
            hid.at[batch, pl.ds(t0 + c * CHUNK, CHUNK)], bufs[q], gsems[q])

    def s_copy(c, q):
        return pltpu.make_async_copy(
            bufs[q], out.at[batch, pl.ds(t0 + c * CHUNK, CHUNK)], ssems[q])

    g_copy(0, 0).start()

    pltpu.sync_copy(idx, idx_v)
    zeros = jnp.zeros((VL,), jnp.float32)
    ones = jnp.ones((VL,), jnp.float32)
    for k in range(TPW // VL):
        mask_v[pl.ds(k * VL, VL)] = zeros
    nidx = idx.shape[0]
    for k in range(nidx // VL):
        iv = idx_v[pl.ds(k * VL, VL)] - t0
        inb = (iv >= 0) & (iv < TPW)
        ivc = jnp.clip(iv, 0, TPW - 1)
        plsc.store_scatter(mask_v, [ivc], ones, mask=inb)

    def chunk_step(c, q):
        qn = (q + 1) % NB

        @pl.when(c >= NB - 1)
        def _():
            s_copy(c - (NB - 1), qn).wait()

        @pl.when(c + 1 < NCH)
        def _():
            g_copy(c + 1, qn).start()

        g_copy(c, q).wait()
        buf = bufs[q]

        def row_step(j, _):
            tloc = c * CHUNK + j
            m = plsc.load_gather(mask_v, [jnp.full((VL,), tloc, jnp.int32)])
            sel = jnp.max(m, axis=0) > 0.5

            @pl.when(sel)
            def _():
                @plsc.parallel_loop(0, F, VL, unroll=16)
                def _(v):
                    sl = pl.ds(v, VL)
                    buf[j, sl] = _gelu_vec(buf[j, sl])

            return 0

        lax.fori_loop(0, CHUNK, row_step, 0)
        s_copy(c, q).start()

    def outer(g, _):
        for k in range(NB):
            chunk_step(g * NB + k, k)
        return 0

    lax.fori_loop(0, NCH // NB, outer, 0)

    for k in range(1, NB):
        s_copy(NCH - NB + k, k).wait()


@functools.partial(
    pl.kernel,
    out_type=jax.ShapeDtypeStruct((B, T, F), jnp.float32),
    mesh=plsc.VectorSubcoreMesh(core_axis_name="c", subcore_axis_name="s"),
    compiler_params=pltpu.CompilerParams(needs_layout_passes=False),
    scratch_types=[
        [pltpu.VMEM((CHUNK, F), jnp.float32) for _ in range(NB)],
        pltpu.VMEM((TPW,), jnp.float32),
        pltpu.VMEM((512,), jnp.int32),
        [pltpu.SemaphoreType.DMA for _ in range(NB)],
        [pltpu.SemaphoreType.DMA for _ in range(NB)],
    ],
)
def _sc_kernel(hid, idx, out, bufs, mask_v, idx_v, gsems, ssems):
    _sc_body(hid, idx, out, bufs, mask_v, idx_v, gsems, ssems)


def kernel(hidden_states, true_indices):
    return _sc_kernel(hidden_states, true_indices)


# speedup vs baseline: 1.1164x; 1.1164x over previous
import functools

import jax
import jax.numpy as jnp
from jax import lax
from jax.experimental import pallas as pl
from jax.experimental.pallas import tpu as pltpu
from jax.experimental.pallas import tpu_sc as plsc

NC = 2
NS = 16
NW = NC * NS

B, T, F = 16, 1024, 768
TPW = T // NC
CHUNK = 32
NCH = TPW // CHUNK
NB = 4
SUB = 8
VL = 16

_K2 = -2.0 * 0.7978845608028654
_A = 0.044715


def _gelu_vec(x):
    x2 = x * x
    arg = (_K2 * x) * (1.0 + _A * x2)
    return x / (1.0 + jnp.exp(arg))


def _sc_body(hid, idx, out, bufs, mask_v, idx_v, gsems, ssems):
    wid = lax.axis_index("s") * NC + lax.axis_index("c")
    batch = wid // NC
    t0 = (wid % NC) * TPW

    def g_copy(c, q):
        return pltpu.make_async_copy(
            hid.at[batch, pl.ds(t0 + c * CHUNK, CHUNK)], bufs[q], gsems[q])

    def s_copy(c, q):
        return pltpu.make_async_copy(
            bufs[q], out.at[batch, pl.ds(t0 + c * CHUNK, CHUNK)], ssems[q])

    g_copy(0, 0).start()

    pltpu.sync_copy(idx, idx_v)
    zeros = jnp.zeros((VL,), jnp.float32)
    ones = jnp.ones((VL,), jnp.float32)
    for k in range(TPW // VL):
        mask_v[pl.ds(k * VL, VL)] = zeros
    nidx = idx.shape[0]
    for k in range(nidx // VL):
        iv = idx_v[pl.ds(k * VL, VL)] - t0
        inb = (iv >= 0) & (iv < TPW)
        ivc = jnp.clip(iv, 0, TPW - 1)
        plsc.store_scatter(mask_v, [ivc], ones, mask=inb)

    def chunk_step(c, q):
        qn = (q + 1) % NB

        @pl.when(c >= NB - 1)
        def _():
            s_copy(c - (NB - 1), qn).wait()

        @pl.when(c + 1 < NCH)
        def _():
            g_copy(c + 1, qn).start()

        g_copy(c, q).wait()
        buf = bufs[q]

        def row_step(j, _):
            tloc = c * CHUNK + j
            m = plsc.load_gather(mask_v, [jnp.full((VL,), tloc, jnp.int32)])
            sel = m[0] > 0.5

            @pl.when(sel)
            def _():
                @plsc.parallel_loop(0, F, VL, unroll=16)
                def _(v):
                    sl = pl.ds(v, VL)
                    buf[j, sl] = _gelu_vec(buf[j, sl])

            return 0

        lax.fori_loop(0, CHUNK, row_step, 0)
        s_copy(c, q).start()

    def outer(g, _):
        for k in range(NB):
            chunk_step(g * NB + k, k)
        return 0

    lax.fori_loop(0, NCH // NB, outer, 0)

    for k in range(1, NB):
        s_copy(NCH - NB + k, k).wait()


@functools.partial(
    pl.kernel,
    out_type=jax.ShapeDtypeStruct((B, T, F), jnp.float32),
    mesh=plsc.VectorSubcoreMesh(core_axis_name="c", subcore_axis_name="s"),
    compiler_params=pltpu.CompilerParams(needs_layout_passes=False),
    scratch_types=[
        [pltpu.VMEM((CHUNK, F), jnp.float32) for _ in range(NB)],
        pltpu.VMEM((TPW,), jnp.float32),
        pltpu.VMEM((512,), jnp.int32),
        [pltpu.SemaphoreType.DMA for _ in range(NB)],
        [pltpu.SemaphoreType.DMA for _ in range(NB)],
    ],
)
def _sc_kernel(hid, idx, out, bufs, mask_v, idx_v, gsems, ssems):
    _sc_body(hid, idx, out, bufs, mask_v, idx_v, gsems, ssems)


def kernel(hidden_states, true_indices):
    return _sc_kernel(hidden_states, true_indices)
